# baseline (speedup 1.0000x reference)
import functools

import jax
import jax.numpy as jnp
from jax import lax
from jax.experimental import pallas as pl
from jax.experimental.pallas import tpu as pltpu
from jax.experimental.pallas import tpu_sc as plsc

B = 4096
F = 26
V = 100000
D = 6
N = B * F
FV = F * V
CH = 128
NW = 32
NCH = N // (NW * CH)
NPW = NCH * CH
TEMP = 0.5
BN_EPS = 1e-3
TARGET_VEC_SIZES = (1, 2, 4, 6)


def _sc_gather(idx, emb_flat, bias_1d):
    mesh = plsc.VectorSubcoreMesh(core_axis_name="c", subcore_axis_name="s")

    @functools.partial(
        pl.kernel,
        out_type=[
            jax.ShapeDtypeStruct((N, 8), jnp.float32),
            jax.ShapeDtypeStruct((N,), jnp.float32),
        ],
        mesh=mesh,
        scratch_types=[
            pltpu.VMEM((NCH, CH), jnp.int32),
            pltpu.VMEM((NPW, 8), jnp.float32),
            pltpu.VMEM((NPW,), jnp.float32),
            pltpu.SemaphoreType.DMA,
            pltpu.SemaphoreType.DMA,
        ],
        compiler_params=pltpu.CompilerParams(use_tc_tiling_on_sc=False),
    )
    def k(idx_hbm, emb_hbm, bias_hbm, out_emb, out_bias,
          idx_v, rows_v, brows_v, sem_e, sem_b):
        w = lax.axis_index("s") * 2 + lax.axis_index("c")
        base = w * NPW
        pltpu.sync_copy(idx_hbm.at[pl.ds(w * NCH, NCH)], idx_v)
        cps = []
        for j in range(NCH):
            sl = pl.ds(j * CH, CH)
            cps.append(pltpu.async_copy(
                emb_hbm.at[idx_v.at[j]], rows_v.at[sl], sem_e))
            cps.append(pltpu.async_copy(
                bias_hbm.at[idx_v.at[j]], brows_v.at[sl], sem_b))
        for cp in cps:
            cp.wait()
        pltpu.sync_copy(rows_v, out_emb.at[pl.ds(base, NPW)])
        pltpu.sync_copy(brows_v, out_bias.at[pl.ds(base, NPW)])

    return k(idx, emb_flat, bias_1d)


def _choice_row(p):
    ki = lax.broadcasted_iota(jnp.int32, (len(TARGET_VEC_SIZES), F * D), 0)
    jd = lax.broadcasted_iota(jnp.int32, (len(TARGET_VEC_SIZES), F * D), 1) % D
    sel = jnp.where(jd < 2, jd, jnp.where(jd < 4, 2, 3))
    K = (sel == ki).astype(jnp.float32)
    S = jnp.dot(p, K, preferred_element_type=jnp.float32)
    fi = lax.broadcasted_iota(jnp.int32, (F, F * D), 0)
    jf = lax.broadcasted_iota(jnp.int32, (F, F * D), 1) // D
    E = (fi == jf).astype(jnp.float32)
    return jnp.sum(S * E, axis=0, keepdims=True)


def _tc_body(x_ref, bv_ref, nas_ref, W1_ref, b1_ref, W2_ref, b2_ref,
             W3_ref, b3_ref, o_ref):
    x = x_ref[...]
    mean = jnp.mean(x, axis=0, keepdims=True)
    var = jnp.mean(x * x, axis=0, keepdims=True) - mean * mean
    inv = lax.rsqrt(var + BN_EPS)
    logits = nas_ref[...] * (1.0 / TEMP)
    m = jnp.max(logits, axis=1, keepdims=True)
    e = jnp.exp(logits - m)
    p = e / jnp.sum(e, axis=1, keepdims=True)
    c = _choice_row(p)
    xs = (x - mean) * (c * inv)
    h = jnp.dot(xs, W1_ref[...], preferred_element_type=jnp.float32)
    h = jnp.maximum(h + b1_ref[...], 0.0)
    h = jnp.dot(h, W2_ref[...], preferred_element_type=jnp.float32)
    h = jnp.maximum(h + b2_ref[...], 0.0)
    o = jnp.dot(h, W3_ref[...], preferred_element_type=jnp.float32)
    o = o + b3_ref[...]
    bsum = jnp.sum(bv_ref[...], axis=1, keepdims=True)
    o_ref[...] = jax.nn.sigmoid(o + bsum)


def _tc_mlp(x, bv, nas, W1, b1, W2, b2, W3, b3):
    return pl.pallas_call(
        _tc_body,
        out_shape=jax.ShapeDtypeStruct((B, 1), jnp.float32),
    )(x, bv, nas, W1, b1, W2, b2, W3, b3)


def kernel(inputs, emb_table, bias_table, nas_logits, W1, b1, W2, b2, W3, b3):
    inputs = inputs.astype(jnp.int32)
    offs = (jnp.arange(F, dtype=jnp.int32) * V)[None, :]
    idx = (inputs + offs).reshape(N // CH, CH)
    emb_flat = jnp.pad(emb_table, ((0, 0), (0, 0), (0, 2))).reshape(FV, 8)
    bias_1d = bias_table.reshape(FV)
    rows, brows = _sc_gather(idx, emb_flat, bias_1d)
    x = rows.reshape(B, F, 8)[:, :, :D].reshape(B, F * D)
    bv = brows.reshape(B, F)
    out = _tc_mlp(x, bv, nas_logits, W1, b1.reshape(1, -1), W2,
                  b2.reshape(1, -1), W3, b3.reshape(1, -1))
    return out.reshape(B)


# speedup vs baseline: 1.1035x; 1.1035x over previous
import functools

import jax
import jax.numpy as jnp
from jax import lax
from jax.experimental import pallas as pl
from jax.experimental.pallas import tpu as pltpu
from jax.experimental.pallas import tpu_sc as plsc

B = 4096
F = 26
V = 100000
D = 6
N = B * F
FV = F * V
CH = 128
NW = 32
NCH = N // (NW * CH)
NPW = NCH * CH
TEMP = 0.5
BN_EPS = 1e-3
TARGET_VEC_SIZES = (1, 2, 4, 6)

FT = 3
VT = 781
NTILE = D * FT * VT
TPW = 440
NTILEP = NW * TPW
OFF2 = NTILEP * 1024
NB = D * 2 * V
NBP = 1200128
OFF3 = OFF2 + NBP
NC = D * 24 * 32
NCP = 5120
ROWS_OUT = (OFF3 + NCP) // 1024
KRING = 8


def _detile(emb_t, b8, c8):
    mesh = plsc.VectorSubcoreMesh(core_axis_name="c", subcore_axis_name="s")

    @functools.partial(
        pl.kernel,
        out_type=jax.ShapeDtypeStruct((ROWS_OUT, 8, 128), jnp.float32),
        mesh=mesh,
        scratch_types=[pltpu.SemaphoreType.DMA],
    )
    def k(src, b_in, c_in, out, sem):
        w = lax.axis_index("s") * 2 + lax.axis_index("c")
        t0 = w * TPW

        def mk(i):
            t = t0 + i
            ts = jnp.minimum(t, NTILE - 1)
            d = ts // (FT * VT)
            r = ts % (FT * VT)
            ft = r // VT
            vt = r % VT
            fo = pl.multiple_of(ft * 8, 8)
            vo = pl.multiple_of(vt * 128, 128)
            return src.at[d, pl.ds(fo, 8), pl.ds(vo, 128)], out.at[t]

        def body(i, c):
            s, dst = mk(i)
            pltpu.async_copy(s, dst, sem)

            @pl.when(i >= KRING)
            def _():
                s2, dst2 = mk(i - KRING)
                pltpu.make_async_copy(s2, dst2, sem).wait()
            return c

        lax.fori_loop(0, TPW, body, 0)

        def drain(i, c):
            s2, dst2 = mk(TPW - KRING + i)
            pltpu.make_async_copy(s2, dst2, sem).wait()
            return c

        lax.fori_loop(0, KRING, drain, 0)

        @pl.when(w == 0)
        def _():
            pltpu.sync_copy(b_in, out.at[pl.ds(NTILEP, NBP // 1024)])
            pltpu.sync_copy(c_in, out.at[pl.ds(NTILEP + NBP // 1024,
                                               NCP // 1024)])

    return k(emb_t, b8, c8)


def _sc_gather(idx6, bidx, emb_1d, bias_1d):
    mesh = plsc.VectorSubcoreMesh(core_axis_name="c", subcore_axis_name="s")

    @functools.partial(
        pl.kernel,
        out_type=[
            jax.ShapeDtypeStruct((D, N), jnp.float32),
            jax.ShapeDtypeStruct((N,), jnp.float32),
        ],
        mesh=mesh,
        scratch_types=[
            pltpu.VMEM((D, NPW), jnp.int32),
            pltpu.VMEM((NCH, CH), jnp.int32),
            pltpu.VMEM((D, NPW), jnp.float32),
            pltpu.VMEM((NPW,), jnp.float32),
            pltpu.SemaphoreType.DMA,
            pltpu.SemaphoreType.DMA,
        ],
        compiler_params=pltpu.CompilerParams(use_tc_tiling_on_sc=False),
    )
    def k(idx_hbm, bidx_hbm, emb_hbm, bias_hbm, out_emb, out_bias,
          idx_v, bidx_v, rows_v, brows_v, sem_e, sem_b):
        w = lax.axis_index("s") * 2 + lax.axis_index("c")
        base = w * NPW
        for d in range(D):
            pltpu.sync_copy(idx_hbm.at[d, pl.ds(base, NPW)], idx_v.at[d])
        pltpu.sync_copy(bidx_hbm.at[pl.ds(w * NCH, NCH)], bidx_v)
        cps = []
        for j in range(NCH):
            sl = pl.ds(j * CH, CH)
            for d in range(D):
                cps.append(pltpu.async_copy(
                    emb_hbm.at[idx_v.at[d, sl]], rows_v.at[d, sl], sem_e))
            cps.append(pltpu.async_copy(
                bias_hbm.at[bidx_v.at[j]], brows_v.at[sl], sem_b))
        for cp in cps:
            cp.wait()
        for d in range(D):
            pltpu.sync_copy(rows_v.at[d], out_emb.at[d, pl.ds(base, NPW)])
        pltpu.sync_copy(brows_v, out_bias.at[pl.ds(base, NPW)])

    return k(idx6, bidx, emb_1d, bias_1d)


def _choice_row(p):
    ki = lax.broadcasted_iota(jnp.int32, (len(TARGET_VEC_SIZES), F * D), 0)
    jd = lax.broadcasted_iota(jnp.int32, (len(TARGET_VEC_SIZES), F * D), 1) % D
    sel = jnp.where(jd < 2, jd, jnp.where(jd < 4, 2, 3))
    K = (sel == ki).astype(jnp.float32)
    S = jnp.dot(p, K, preferred_element_type=jnp.float32)
    fi = lax.broadcasted_iota(jnp.int32, (F, F * D), 0)
    jf = lax.broadcasted_iota(jnp.int32, (F, F * D), 1) // D
    E = (fi == jf).astype(jnp.float32)
    return jnp.sum(S * E, axis=0, keepdims=True)


def _tc_body(x_ref, bv_ref, nas_ref, W1_ref, b1_ref, W2_ref, b2_ref,
             W3_ref, b3_ref, o_ref):
    x = x_ref[...]
    mean = jnp.mean(x, axis=0, keepdims=True)
    var = jnp.mean(x * x, axis=0, keepdims=True) - mean * mean
    inv = lax.rsqrt(var + BN_EPS)
    logits = nas_ref[...] * (1.0 / TEMP)
    m = jnp.max(logits, axis=1, keepdims=True)
    e = jnp.exp(logits - m)
    p = e / jnp.sum(e, axis=1, keepdims=True)
    c = _choice_row(p)
    xs = (x - mean) * (c * inv)
    h = jnp.dot(xs, W1_ref[...], preferred_element_type=jnp.float32)
    h = jnp.maximum(h + b1_ref[...], 0.0)
    h = jnp.dot(h, W2_ref[...], preferred_element_type=jnp.float32)
    h = jnp.maximum(h + b2_ref[...], 0.0)
    o = jnp.dot(h, W3_ref[...], preferred_element_type=jnp.float32)
    o = o + b3_ref[...]
    bsum = jnp.sum(bv_ref[...], axis=1, keepdims=True)
    o_ref[...] = jax.nn.sigmoid(o + bsum)


def _tc_mlp(x, bv, nas, W1, b1, W2, b2, W3, b3):
    return pl.pallas_call(
        _tc_body,
        out_shape=jax.ShapeDtypeStruct((B, 1), jnp.float32),
    )(x, bv, nas, W1, b1, W2, b2, W3, b3)


def kernel(inputs, emb_table, bias_table, nas_logits, W1, b1, W2, b2, W3, b3):
    inputs = inputs.astype(jnp.int32)
    emb_t = jnp.transpose(emb_table, (2, 0, 1))
    bsrc = lax.slice(emb_t, (0, 24, 0), (D, F, V)).reshape(NB)
    b8 = jnp.pad(bsrc, (0, NBP - NB)).reshape(NBP // 1024, 8, 128)
    csrc = lax.slice(emb_t, (0, 0, V - 32), (D, 24, V)).reshape(NC)
    c8 = jnp.pad(csrc, (0, NCP - NC)).reshape(NCP // 1024, 8, 128)
    table = _detile(emb_t, b8, c8).reshape(ROWS_OUT * 1024)
    bias_1d = bias_table.reshape(FV)

    f = jnp.arange(F, dtype=jnp.int32)[None, :]
    v = inputs
    dn = jnp.arange(D, dtype=jnp.int32)[:, None, None]
    ft = f // 8
    fi8 = f % 8
    vt = v // 128
    vi = v % 128
    ea = ((dn * FT + ft[None]) * VT + vt[None]) * 1024 + fi8[None] * 128 + vi[None]
    eb = OFF2 + (dn * 2 + (f[None] - 24)) * V + v[None]
    ec = OFF3 + (dn * 24 + f[None]) * 32 + (v[None] - (V - 32))
    eidx = jnp.where(f[None] >= 24, eb, jnp.where(v[None] >= V - 32, ec, ea))
    idx6 = eidx.reshape(D, N)
    bidx = (v + f * V).reshape(N // CH, CH)

    rows6, brows = _sc_gather(idx6, bidx, table, bias_1d)
    x = rows6.reshape(D, B, F).transpose(1, 2, 0).reshape(B, F * D)
    bv = brows.reshape(B, F)
    out = _tc_mlp(x, bv, nas_logits, W1, b1.reshape(1, -1), W2,
                  b2.reshape(1, -1), W3, b3.reshape(1, -1))
    return out.reshape(B)


# speedup vs baseline: 5.8200x; 5.2739x over previous
import functools

import jax
import jax.numpy as jnp
from jax import lax
from jax.experimental import pallas as pl
from jax.experimental.pallas import tpu as pltpu
from jax.experimental.pallas import tpu_sc as plsc

B = 4096
F = 26
V = 100000
D = 6
N = B * F
FV = F * V
CH = 128
NW = 32
NCH = N // (NW * CH)
NPW = NCH * CH
TEMP = 0.5
BN_EPS = 1e-3
TARGET_VEC_SIZES = (1, 2, 4, 6)

FT = 3
VT = 781
NTILE = D * FT * VT
TPW = 440
NTILEP = NW * TPW
OFF2 = NTILEP * 1024
NB = D * 2 * V
NBP = 1200128
OFF3 = OFF2 + NBP
NC = D * 24 * 32
NCP = 5120
ROWS_OUT = (OFF3 + NCP) // 1024
GDET = 55


def _detile(emb_t, b8, c8):
    mesh = plsc.VectorSubcoreMesh(core_axis_name="c", subcore_axis_name="s")

    @functools.partial(
        pl.kernel,
        out_type=jax.ShapeDtypeStruct((ROWS_OUT, 8, 128), jnp.float32),
        mesh=mesh,
        scratch_types=[
            pltpu.VMEM((GDET, 8, 128), jnp.float32),
            pltpu.SemaphoreType.DMA,
        ],
    )
    def k(src, b_in, c_in, out, vbuf, sem):
        w = lax.axis_index("s") * 2 + lax.axis_index("c")
        t0 = w * TPW

        def srcsl(t):
            ts = jnp.minimum(t, NTILE - 1)
            d = ts // (FT * VT)
            r = ts % (FT * VT)
            ft = r // VT
            vt = r % VT
            fo = pl.multiple_of(ft * 8, 8)
            vo = pl.multiple_of(vt * 128, 128)
            return src.at[d, pl.ds(fo, 8), pl.ds(vo, 128)]

        for g in range(TPW // GDET):
            t0g = t0 + g * GDET

            def fire(i, c):
                pltpu.async_copy(srcsl(t0g + i), vbuf.at[i], sem)
                return c

            lax.fori_loop(0, GDET, fire, 0)

            def drain(i, c):
                pltpu.make_async_copy(srcsl(t0g + i), vbuf.at[i], sem).wait()
                return c

            lax.fori_loop(0, GDET, drain, 0)
            pltpu.sync_copy(vbuf, out.at[pl.ds(t0g, GDET)])

        @pl.when(w == 0)
        def _():
            pltpu.sync_copy(b_in, out.at[pl.ds(NTILEP, NBP // 1024)])
            pltpu.sync_copy(c_in, out.at[pl.ds(NTILEP + NBP // 1024,
                                               NCP // 1024)])

    return k(emb_t, b8, c8)


def _sc_gather(idx6, bidx, emb_1d, bias_1d):
    mesh = plsc.VectorSubcoreMesh(core_axis_name="c", subcore_axis_name="s")

    @functools.partial(
        pl.kernel,
        out_type=[
            jax.ShapeDtypeStruct((D, N), jnp.float32),
            jax.ShapeDtypeStruct((N,), jnp.float32),
        ],
        mesh=mesh,
        scratch_types=[
            pltpu.VMEM((D, NPW), jnp.int32),
            pltpu.VMEM((NCH, CH), jnp.int32),
            pltpu.VMEM((D, NPW), jnp.float32),
            pltpu.VMEM((NPW,), jnp.float32),
            pltpu.SemaphoreType.DMA,
            pltpu.SemaphoreType.DMA,
        ],
        compiler_params=pltpu.CompilerParams(use_tc_tiling_on_sc=False),
    )
    def k(idx_hbm, bidx_hbm, emb_hbm, bias_hbm, out_emb, out_bias,
          idx_v, bidx_v, rows_v, brows_v, sem_e, sem_b):
        w = lax.axis_index("s") * 2 + lax.axis_index("c")
        base = w * NPW
        for d in range(D):
            pltpu.sync_copy(idx_hbm.at[d, pl.ds(base, NPW)], idx_v.at[d])
        pltpu.sync_copy(bidx_hbm.at[pl.ds(w * NCH, NCH)], bidx_v)
        cps = []
        for j in range(NCH):
            sl = pl.ds(j * CH, CH)
            for d in range(D):
                cps.append(pltpu.async_copy(
                    emb_hbm.at[idx_v.at[d, sl]], rows_v.at[d, sl], sem_e))
            cps.append(pltpu.async_copy(
                bias_hbm.at[bidx_v.at[j]], brows_v.at[sl], sem_b))
        for cp in cps:
            cp.wait()
        for d in range(D):
            pltpu.sync_copy(rows_v.at[d], out_emb.at[d, pl.ds(base, NPW)])
        pltpu.sync_copy(brows_v, out_bias.at[pl.ds(base, NPW)])

    return k(idx6, bidx, emb_1d, bias_1d)


def _choice_row(p):
    ki = lax.broadcasted_iota(jnp.int32, (len(TARGET_VEC_SIZES), F * D), 0)
    jd = lax.broadcasted_iota(jnp.int32, (len(TARGET_VEC_SIZES), F * D), 1) % D
    sel = jnp.where(jd < 2, jd, jnp.where(jd < 4, 2, 3))
    K = (sel == ki).astype(jnp.float32)
    S = jnp.dot(p, K, preferred_element_type=jnp.float32)
    fi = lax.broadcasted_iota(jnp.int32, (F, F * D), 0)
    jf = lax.broadcasted_iota(jnp.int32, (F, F * D), 1) // D
    E = (fi == jf).astype(jnp.float32)
    return jnp.sum(S * E, axis=0, keepdims=True)


def _tc_body(x_ref, bv_ref, nas_ref, W1_ref, b1_ref, W2_ref, b2_ref,
             W3_ref, b3_ref, o_ref):
    x = x_ref[...]
    mean = jnp.mean(x, axis=0, keepdims=True)
    var = jnp.mean(x * x, axis=0, keepdims=True) - mean * mean
    inv = lax.rsqrt(var + BN_EPS)
    logits = nas_ref[...] * (1.0 / TEMP)
    m = jnp.max(logits, axis=1, keepdims=True)
    e = jnp.exp(logits - m)
    p = e / jnp.sum(e, axis=1, keepdims=True)
    c = _choice_row(p)
    xs = (x - mean) * (c * inv)
    h = jnp.dot(xs, W1_ref[...], preferred_element_type=jnp.float32)
    h = jnp.maximum(h + b1_ref[...], 0.0)
    h = jnp.dot(h, W2_ref[...], preferred_element_type=jnp.float32)
    h = jnp.maximum(h + b2_ref[...], 0.0)
    o = jnp.dot(h, W3_ref[...], preferred_element_type=jnp.float32)
    o = o + b3_ref[...]
    bsum = jnp.sum(bv_ref[...], axis=1, keepdims=True)
    o_ref[...] = jax.nn.sigmoid(o + bsum)


def _tc_mlp(x, bv, nas, W1, b1, W2, b2, W3, b3):
    return pl.pallas_call(
        _tc_body,
        out_shape=jax.ShapeDtypeStruct((B, 1), jnp.float32),
    )(x, bv, nas, W1, b1, W2, b2, W3, b3)


def kernel(inputs, emb_table, bias_table, nas_logits, W1, b1, W2, b2, W3, b3):
    inputs = inputs.astype(jnp.int32)
    emb_t = jnp.transpose(emb_table, (2, 0, 1))
    bsrc = lax.slice(emb_t, (0, 24, 0), (D, F, V)).reshape(NB)
    b8 = jnp.pad(bsrc, (0, NBP - NB)).reshape(NBP // 1024, 8, 128)
    csrc = lax.slice(emb_t, (0, 0, V - 32), (D, 24, V)).reshape(NC)
    c8 = jnp.pad(csrc, (0, NCP - NC)).reshape(NCP // 1024, 8, 128)
    table = _detile(emb_t, b8, c8).reshape(ROWS_OUT * 1024)
    bias_1d = bias_table.reshape(FV)

    f = jnp.arange(F, dtype=jnp.int32)[None, :]
    v = inputs
    dn = jnp.arange(D, dtype=jnp.int32)[:, None, None]
    ft = f // 8
    fi8 = f % 8
    vt = v // 128
    vi = v % 128
    ea = ((dn * FT + ft[None]) * VT + vt[None]) * 1024 + fi8[None] * 128 + vi[None]
    eb = OFF2 + (dn * 2 + (f[None] - 24)) * V + v[None]
    ec = OFF3 + (dn * 24 + f[None]) * 32 + (v[None] - (V - 32))
    eidx = jnp.where(f[None] >= 24, eb, jnp.where(v[None] >= V - 32, ec, ea))
    idx6 = eidx.reshape(D, N)
    bidx = (v + f * V).reshape(N // CH, CH)

    rows6, brows = _sc_gather(idx6, bidx, table, bias_1d)
    x = rows6.reshape(D, B, F).transpose(1, 2, 0).reshape(B, F * D)
    bv = brows.reshape(B, F)
    out = _tc_mlp(x, bv, nas_logits, W1, b1.reshape(1, -1), W2,
                  b2.reshape(1, -1), W3, b3.reshape(1, -1))
    return out.reshape(B)
